# baseline (speedup 1.0000x reference)
import functools

import jax
import jax.numpy as jnp
from jax import lax
from jax.experimental import pallas as pl
from jax.experimental.pallas import tpu as pltpu
from jax.experimental.pallas import tpu_sc as plsc

W_DIRECT = 1.0
W_MULTIPLE = 1.0
W_BACKGROUND = 1.0

NC = 2
NS = 16
L = 16
NW = NC * NS

BIG = 1e30
BIGH = 1e29
IBIG = 2**30


def _make_sc_kernel(N, NPAD, M, MPAD, H, W, KROWS):
    HW = H * W
    WSHIFT = W.bit_length() - 1
    assert W == (1 << WSHIFT)
    KSHIFT = 11
    CMASK = (1 << KSHIFT) - 1
    assert NPAD // L <= CMASK + 1
    assert 2 * (W - 1) * (W - 1) + 1 << KSHIFT < IBIG
    CHUNKS = NPAD // L
    REAL_CHUNKS = N // L
    KMAX = (M + NW - 1) // NW
    mesh = plsc.VectorSubcoreMesh(core_axis_name="c", subcore_axis_name="s")

    @functools.partial(
        pl.kernel,
        mesh=mesh,
        compiler_params=pltpu.CompilerParams(needs_layout_passes=False),
        out_type=[
            jax.ShapeDtypeStruct((NW, KROWS, L), jnp.float32),
            jax.ShapeDtypeStruct((NW, KROWS, L), jnp.float32),
            jax.ShapeDtypeStruct((NW, KROWS, L), jnp.int32),
            jax.ShapeDtypeStruct((NW, NPAD), jnp.float32),
        ],
        scratch_types=[
            pltpu.VMEM((NPAD,), jnp.int32),
            pltpu.VMEM((NPAD,), jnp.int32),
            pltpu.VMEM((NPAD,), jnp.int32),
            pltpu.VMEM((NPAD,), jnp.float32),
            pltpu.VMEM((NPAD,), jnp.float32),
            pltpu.VMEM((NPAD,), jnp.float32),
            pltpu.VMEM((NW, L), jnp.int32),
            pltpu.VMEM((KROWS, L), jnp.float32),
            pltpu.VMEM((KROWS, L), jnp.float32),
            pltpu.VMEM((KROWS, L), jnp.int32),
            pltpu.VMEM((L,), jnp.int32),
            pltpu.VMEM((L,), jnp.int32),
            pltpu.VMEM((L,), jnp.float32),
            pltpu.SemaphoreType.DMA,
            pltpu.SemaphoreType.DMA,
        ],
    )
    def sc_kernel(masks_hbm, ux_hbm, uy_hbm, v_hbm,
                  cnt_o, d2_o, idx_o, any_o,
                  pbase_v, offs0_v, offs1_v, vals0_v, vals1_v,
                  any_v, v_v, cntb, d2b, idxb, tmpd, tmpi, tmpc,
                  sem0, sem1):
        wid = lax.axis_index("s") * NC + lax.axis_index("c")
        pltpu.sync_copy(ux_hbm, vals0_v)
        pltpu.sync_copy(uy_hbm, vals1_v)
        pltpu.sync_copy(v_hbm, v_v)

        zero16f = jnp.zeros((L,), jnp.float32)
        zero16i = jnp.zeros((L,), jnp.int32)

        def init_body(i):
            sl = pl.ds(i * L, L)
            any_v[sl] = zero16f
            xi = jnp.clip(vals0_v[sl].astype(jnp.int32), 0, W - 1)
            yi = jnp.clip(vals1_v[sl].astype(jnp.int32), 0, H - 1)
            pbase_v[sl] = yi * W + xi
        plsc.parallel_loop(0, CHUNKS, unroll=8)(init_body)

        for k in range(KROWS):
            cntb[k, :] = zero16f
            d2b[k, :] = zero16f
            idxb[k, :] = zero16i

        base_iota = lax.iota(jnp.int32, L)
        vvec = v_v[wid, :]
        offs_bufs = (offs0_v, offs1_v)
        vals_bufs = (vals0_v, vals1_v)
        sems = (sem0, sem1)

        def build_offs(buf, mhw):
            def offs_body(i):
                sl = pl.ds(i * L, L)
                buf[sl] = pbase_v[sl] + mhw
            plsc.parallel_loop(0, CHUNKS, unroll=8)(offs_body)

        def issue(k):
            b = k % 2
            m = k * NW + wid

            @pl.when(m < M)
            def _():
                build_offs(offs_bufs[b], m * HW)
                pltpu.async_copy(
                    masks_hbm.at[offs_bufs[b]], vals_bufs[b], sems[b])

        def process(k):
            b = k % 2
            m = k * NW + wid

            @pl.when(m < M)
            def _():
                pltpu.make_async_copy(
                    masks_hbm.at[offs_bufs[b]], vals_bufs[b], sems[b]).wait()
                vals_v = vals_bufs[b]
                vxi = vvec[k]
                vyi = vvec[KROWS + k]

                def chunk_loop(i, carry):
                    bkey, cnt = carry
                    sl = pl.ds(i * L, L)
                    val = vals_v[sl]
                    member = val > 0.0
                    p = pbase_v[sl]
                    xi = jnp.bitwise_and(p, W - 1)
                    yi = lax.shift_right_logical(p, WSHIFT)
                    dxi = xi - vxi
                    dyi = yi - vyi
                    d2i = dxi * dxi + dyi * dyi
                    key = jnp.bitwise_or(lax.shift_left(d2i, KSHIFT), i)
                    key = jnp.where(member, key, IBIG)
                    bkey = jnp.minimum(bkey, key)
                    mf = jnp.where(member, 1.0, 0.0)
                    cnt = cnt + mf
                    any_v[sl] = jnp.maximum(any_v[sl], mf)
                    return (bkey, cnt)

                bkey, cnt = plsc.parallel_loop(
                    0, REAL_CHUNKS, unroll=8,
                    carry=(jnp.full((L,), IBIG, jnp.int32), zero16f),
                )(chunk_loop)

                tmpd[:] = bkey
                tmpi[:] = base_iota
                tmpc[:] = cnt
                for sh in (8, 4, 2, 1):
                    perm = jnp.bitwise_xor(base_iota, sh)
                    okey = plsc.load_gather(tmpd, [perm])
                    og = plsc.load_gather(tmpi, [perm])
                    oc = plsc.load_gather(tmpc, [perm])
                    keyv = tmpd[:]
                    gv = tmpi[:]
                    better = (okey < keyv) | ((okey == keyv) & (og < gv))
                    tmpd[:] = jnp.where(better, okey, keyv)
                    tmpi[:] = jnp.where(better, og, gv)
                    tmpc[:] = tmpc[:] + oc
                keyv = tmpd[:]
                cntb[k, :] = tmpc[:]
                d2b[k, :] = lax.shift_right_logical(keyv, KSHIFT).astype(
                    jnp.float32)
                idxb[k, :] = jnp.bitwise_and(keyv, CMASK) * L + tmpi[:]

        issue(0)
        for k in range(KMAX):
            if k + 1 < KMAX:
                issue(k + 1)
            process(k)

        pltpu.sync_copy(cntb, cnt_o.at[wid])
        pltpu.sync_copy(d2b, d2_o.at[wid])
        pltpu.sync_copy(idxb, idx_o.at[wid])
        pltpu.sync_copy(any_v, any_o.at[wid])

    return sc_kernel


def _make_linearizer(MS, H, W):
    HW = H * W

    def lin_body(m_ref, o_ref):
        o_ref[...] = m_ref[...].reshape(HW)

    return pl.pallas_call(
        lin_body,
        grid=(MS,),
        in_specs=[pl.BlockSpec((1, H, W), lambda m: (m, 0, 0))],
        out_specs=pl.BlockSpec((HW,), lambda m: (m,)),
        out_shape=jax.ShapeDtypeStruct((MS * HW,), jnp.float32),
    )


def _make_tc_kernel(N, NPAD, M, MPAD, NROW, NANY):
    NCOL = NPAD // NROW

    def tc_body(cnt_s, vx_s, vy_s, cnt_v, d2_v, idx_v, any_r, ux_r, uy_r,
                pairs_o, cost_o):
        ux = ux_r[...]
        uy = uy_r[...]
        rr = lax.broadcasted_iota(jnp.int32, (NROW, NCOL), 0)
        cc = lax.broadcasted_iota(jnp.int32, (NROW, NCOL), 1)
        nidx = rr * NCOL + cc

        anyacc = any_r[0]
        for i in range(1, NANY):
            anyacc = anyacc + any_r[i]
        avail = jnp.where((anyacc == 0.0) & (nidx < N), 1.0, 0.0)

        miota = lax.broadcasted_iota(jnp.int32, (1, MPAD), 1)
        cntv = cnt_v[...]
        validm = (cntv > 0.0) & (miota < M)
        wgt = jnp.where(cntv == 1.0, W_DIRECT, W_MULTIPLE).astype(jnp.float32)
        cost0 = jnp.sum(jnp.where(validm, jnp.sqrt(d2_v[...]) * wgt, 0.0))
        pairs = jnp.where(validm, idx_v[...], -1)
        nempty = jnp.sum(jnp.where((cntv <= 0.0) & (miota < M), 1, 0))

        def bg_body(j, st):
            availf, prs, cst = st
            cj = cnt_s[j]
            vxj = vx_s[j]
            vyj = vy_s[j]
            dx = ux - vxj
            dy = uy - vyj
            ds2 = dx * dx + dy * dy
            cand = jnp.where(availf > 0.0, ds2, BIG)
            mv = jnp.min(cand)
            do = (cj == 0.0) & (mv < BIGH)
            mi = jnp.min(jnp.where(cand == mv, nidx, IBIG))
            prs = jnp.where((miota == j) & do, mi, prs)
            cst = cst + jnp.where(do, jnp.sqrt(mv) * W_BACKGROUND, 0.0)
            availf = jnp.where((nidx == mi) & do, 0.0, availf)
            return (availf, prs, cst)

        st = (avail, pairs, jnp.float32(0.0))
        _, pairs, bgcost = lax.cond(
            nempty > 0, lambda s: lax.fori_loop(0, M, bg_body, s),
            lambda s: s, st)

        pairs_o[...] = pairs
        cost_o[...] = (cost0 + bgcost).reshape(1, 1)

    smem = pl.BlockSpec(memory_space=pltpu.SMEM)
    vmem = pl.BlockSpec(memory_space=pltpu.VMEM)
    return pl.pallas_call(
        tc_body,
        in_specs=[smem, smem, smem, vmem, vmem, vmem, vmem, vmem, vmem],
        out_specs=[vmem, vmem],
        out_shape=[
            jax.ShapeDtypeStruct((1, MPAD), jnp.int32),
            jax.ShapeDtypeStruct((1, 1), jnp.float32),
        ],
    )


def kernel(pred_points, points, masks):
    N = pred_points.shape[0]
    M, H, W = masks.shape

    NPAD = ((N + NW * L - 1) // (NW * L)) * (NW * L)
    MPAD = 256
    NROW = 8
    NSPLIT = 2
    MS = M // NSPLIT
    KROWS = 8

    ux = jnp.pad(pred_points[:, 0], (0, NPAD - N))
    uy = jnp.pad(pred_points[:, 1], (0, NPAD - N))
    v2 = jnp.pad(points, ((0, MPAD - M), (0, 0)))

    lin = _make_linearizer(MS, H, W)
    sc = _make_sc_kernel(N, NPAD, MS, MPAD, H, W, KROWS)

    cnt_parts, d2_parts, idx_parts, any_parts = [], [], [], []
    for s in range(NSPLIT):
        vs = points[s * MS:(s + 1) * MS]
        vsp = jnp.pad(vs, ((0, KROWS * NW - MS), (0, 0)))
        vw = jnp.concatenate(
            [vsp[:, 0].reshape(KROWS, NW).T, vsp[:, 1].reshape(KROWS, NW).T],
            axis=1).astype(jnp.int32)
        masks_lin = lin(masks[s * MS:(s + 1) * MS])
        cnt3, d23, idx3, any2 = sc(masks_lin, ux, uy, vw)
        cnt_parts.append(cnt3[:, :, 0].T.reshape(-1)[:MS])
        d2_parts.append(d23[:, :, 0].T.reshape(-1)[:MS])
        idx_parts.append(idx3[:, :, 0].T.reshape(-1)[:MS])
        any_parts.append(any2)

    cnt_m = jnp.pad(jnp.concatenate(cnt_parts), (0, MPAD - M))
    d2_m = jnp.pad(jnp.concatenate(d2_parts), (0, MPAD - M))
    idx_m = jnp.pad(jnp.concatenate(idx_parts), (0, MPAD - M))
    any_all = jnp.concatenate(any_parts, axis=0)

    tc = _make_tc_kernel(N, NPAD, M, MPAD, NROW, NSPLIT * NW)
    pairs_v, cost = tc(
        cnt_m, v2[:, 0], v2[:, 1],
        cnt_m.reshape(1, MPAD), d2_m.reshape(1, MPAD), idx_m.reshape(1, MPAD),
        any_all.reshape(NSPLIT * NW, NROW, NPAD // NROW),
        ux.reshape(NROW, NPAD // NROW), uy.reshape(NROW, NPAD // NROW))

    pairs_arr = jnp.stack(
        [pairs_v[0, :M], jnp.arange(M, dtype=jnp.int32)], axis=1)
    return (pairs_arr, cost[0, 0])


# speedup vs baseline: 2.3377x; 2.3377x over previous
import functools

import jax
import jax.numpy as jnp
from jax import lax
from jax.experimental import pallas as pl
from jax.experimental.pallas import tpu as pltpu
from jax.experimental.pallas import tpu_sc as plsc

W_DIRECT = 1.0
W_MULTIPLE = 1.0
W_BACKGROUND = 1.0

NC = 2
NS = 16
L = 16
NW = NC * NS

BIG = 1e30
BIGH = 1e29
IBIG = 2**30


def _make_pack_kernel(M, H, W, GW):
    def pack_body(m_ref, p_ref, a_ref):
        rows = lax.broadcasted_iota(jnp.int32, (GW, M), 0)
        cols = lax.broadcasted_iota(jnp.int32, (GW, M), 1)
        sel = (cols >= rows * L) & (cols < rows * L + L)
        sh = jnp.clip(cols - rows * L, 0, L - 1)
        wmat = jnp.where(
            sel, jnp.left_shift(1, sh).astype(jnp.float32), 0.0)
        for r in range(8):
            acc = jax.lax.dot_general(
                wmat, m_ref[:, r, :], (((1,), (0,)), ((), ())),
                preferred_element_type=jnp.float32,
                precision=jax.lax.Precision.DEFAULT)
            anyrow = jnp.where(jnp.sum(acc, axis=0) > 0.0, 1.0, 0.0)
            p_ref[:, r * W:(r + 1) * W] = acc
            a_ref[r * W:(r + 1) * W] = anyrow

    return pl.pallas_call(
        pack_body,
        grid=(H // 8,),
        in_specs=[pl.BlockSpec((M, 8, W), lambda y: (0, y, 0))],
        out_specs=[
            pl.BlockSpec((GW, 8 * W), lambda y: (0, y)),
            pl.BlockSpec((8 * W,), lambda y: (y,)),
        ],
        out_shape=[
            jax.ShapeDtypeStruct((GW, H * W), jnp.float32),
            jax.ShapeDtypeStruct((H * W,), jnp.float32),
        ],
    )


def _make_sc_kernel(N, NPAD, MG, H, W, G, GW):
    HW = H * W
    WSHIFT = W.bit_length() - 1
    assert W == (1 << WSHIFT)
    PPW = NPAD // NW
    PCH = PPW // L
    assert PCH <= 64
    KSHIFT = 6
    mesh = plsc.VectorSubcoreMesh(core_axis_name="c", subcore_axis_name="s")

    @functools.partial(
        pl.kernel,
        mesh=mesh,
        compiler_params=pltpu.CompilerParams(needs_layout_passes=False),
        out_type=[
            jax.ShapeDtypeStruct((NW, MG, L), jnp.int32),
            jax.ShapeDtypeStruct((NW, MG, L), jnp.int32),
            jax.ShapeDtypeStruct((NPAD,), jnp.float32),
        ],
        scratch_types=[
            pltpu.VMEM((PPW,), jnp.float32),
            pltpu.VMEM((PPW,), jnp.float32),
            pltpu.VMEM((PPW,), jnp.int32),
            pltpu.VMEM((PPW,), jnp.int32),
            pltpu.VMEM((PPW,), jnp.float32),
            pltpu.VMEM((PPW,), jnp.float32),
            pltpu.VMEM((PPW,), jnp.int32),
            pltpu.VMEM((PPW,), jnp.int32),
            pltpu.VMEM((G, L), jnp.int32),
            pltpu.VMEM((G, L), jnp.int32),
            pltpu.VMEM((MG, L), jnp.int32),
            pltpu.VMEM((MG, L), jnp.int32),
            pltpu.SemaphoreType.DMA,
            pltpu.SemaphoreType.DMA,
        ],
    )
    def sc_kernel(packed_hbm, anyg_hbm, ux_hbm, uy_hbm, vx_hbm, vy_hbm,
                  keys_o, cnt_o, any_o,
                  tmpx, tmpy, pidx_v, offs_v, colv, anyv_v,
                  xv, yv, vgx, vgy, keysb, cntb, sem0, sem1):
        wid = lax.axis_index("s") * NC + lax.axis_index("c")
        base = wid * PPW
        pltpu.sync_copy(ux_hbm.at[pl.ds(base, PPW)], tmpx)
        pltpu.sync_copy(uy_hbm.at[pl.ds(base, PPW)], tmpy)
        pltpu.sync_copy(vx_hbm, vgx)
        pltpu.sync_copy(vy_hbm, vgy)

        def init_body(i):
            sl = pl.ds(i * L, L)
            xi = jnp.clip(tmpx[sl].astype(jnp.int32), 0, W - 1)
            yi = jnp.clip(tmpy[sl].astype(jnp.int32), 0, H - 1)
            xv[sl] = xi
            yv[sl] = yi
            pidx_v[sl] = yi * W + xi
        plsc.parallel_loop(0, PCH, unroll=8)(init_body)

        pltpu.async_copy(anyg_hbm.at[pidx_v], anyv_v, sem1).wait()

        nreal = jnp.maximum(jnp.minimum(N - base, PPW), 0)
        biota = lax.iota(jnp.int32, L)

        def pad_any(i, c):
            sl = pl.ds(i * L, L)
            pt = biota + i * L
            anyv_v[sl] = jnp.where(pt < nreal, anyv_v[sl], 1.0)
            return c
        lax.fori_loop(nreal // L, PCH, pad_any, 0)

        pltpu.sync_copy(anyv_v, any_o.at[pl.ds(base, PPW)])

        base_iota = lax.iota(jnp.int32, L)

        def group_body(g, carry):
            vgxr = vgx[g, :]
            vgyr = vgy[g, :]

            goff = g * HW

            def offs_body(i):
                sl = pl.ds(i * L, L)
                offs_v[sl] = pidx_v[sl] + goff
            plsc.parallel_loop(0, PCH, unroll=8)(offs_body)
            pltpu.async_copy(packed_hbm.at[offs_v], colv, sem0).wait()

            def padc(i, c):
                sl = pl.ds(i * L, L)
                pt = biota + i * L
                colv[sl] = jnp.where(pt < nreal, colv[sl], 0.0)
                return c
            lax.fori_loop(nreal // L, PCH, padc, 0)

            for j in range(L):
                vxj = vgxr[j]
                vyj = vgyr[j]

                def chunk_body(c, cr):
                    bkey, cnt = cr
                    sl = pl.ds(c * L, L)
                    wi = colv[sl].astype(jnp.int32)
                    bit = jnp.bitwise_and(
                        lax.shift_right_logical(wi, j), 1)
                    dx = xv[sl] - vxj
                    dy = yv[sl] - vyj
                    d2 = dx * dx + dy * dy
                    key = jnp.bitwise_or(lax.shift_left(d2, KSHIFT), c)
                    key = jnp.where(bit > 0, key, IBIG)
                    bkey = jnp.minimum(bkey, key)
                    cnt = cnt + bit
                    return (bkey, cnt)

                bkey, cnt = plsc.parallel_loop(
                    0, PCH, unroll=8,
                    carry=(jnp.full((L,), IBIG, jnp.int32),
                           jnp.zeros((L,), jnp.int32)),
                )(chunk_body)
                keysb[g * L + j, :] = bkey
                cntb[g * L + j, :] = cnt
            return carry
        lax.fori_loop(0, G, group_body, 0)

        pltpu.sync_copy(keysb, keys_o.at[wid])
        pltpu.sync_copy(cntb, cnt_o.at[wid])

    return sc_kernel


def _make_tc_reduce(N, NPAD, MG, MPAD, NROW, KSHIFT, PPW):
    NCOL = NPAD // NROW

    def red_body(keys_r, cnt_r, any_r, cntv_o, d2v_o, idxv_o, avail_o):
        keys = keys_r[...]
        cnts = cnt_r[...]
        i0 = lax.broadcasted_iota(jnp.int32, (NW * L, MG), 0)
        w = i0 // L
        lane = i0 - w * L
        d2 = lax.shift_right_logical(keys, KSHIFT)
        c = jnp.bitwise_and(keys, (1 << KSHIFT) - 1)
        gidx = w * PPW + c * L + lane
        mind2 = jnp.min(d2, axis=0, keepdims=True)
        idxc = jnp.where(d2 == mind2, gidx, IBIG)
        minidx = jnp.min(idxc, axis=0, keepdims=True)
        cntm = jnp.sum(cnts, axis=0, keepdims=True)

        pad = jnp.zeros((1, MPAD - MG), jnp.float32)
        ipad = jnp.zeros((1, MPAD - MG), jnp.int32)
        cntv_o[...] = jnp.concatenate(
            [cntm.astype(jnp.float32), pad], axis=1)
        d2v_o[...] = jnp.concatenate(
            [mind2.astype(jnp.float32), pad], axis=1)
        idxv_o[...] = jnp.concatenate([minidx, ipad], axis=1)

        anyv = any_r[...]
        rr = lax.broadcasted_iota(jnp.int32, (NROW, NCOL), 0)
        cc = lax.broadcasted_iota(jnp.int32, (NROW, NCOL), 1)
        nidx = rr * NCOL + cc
        avail_o[...] = jnp.where((anyv == 0.0) & (nidx < N), 1.0, 0.0)

    vmem = pl.BlockSpec(memory_space=pltpu.VMEM)
    return pl.pallas_call(
        red_body,
        in_specs=[vmem, vmem, vmem],
        out_specs=[vmem, vmem, vmem, vmem],
        out_shape=[
            jax.ShapeDtypeStruct((1, MPAD), jnp.float32),
            jax.ShapeDtypeStruct((1, MPAD), jnp.float32),
            jax.ShapeDtypeStruct((1, MPAD), jnp.int32),
            jax.ShapeDtypeStruct((NROW, NCOL), jnp.float32),
        ],
    )


def _make_tc_matcher(N, NPAD, M, MPAD, NROW):
    NCOL = NPAD // NROW

    def tc_body(cnt_s, vx_s, vy_s, cnt_v, d2_v, idx_v, avail_r, ux_r, uy_r,
                pairs_o, cost_o):
        ux = ux_r[...]
        uy = uy_r[...]
        rr = lax.broadcasted_iota(jnp.int32, (NROW, NCOL), 0)
        cc = lax.broadcasted_iota(jnp.int32, (NROW, NCOL), 1)
        nidx = rr * NCOL + cc
        avail = avail_r[...]

        miota = lax.broadcasted_iota(jnp.int32, (1, MPAD), 1)
        cntv = cnt_v[...]
        validm = (cntv > 0.0) & (miota < M)
        wgt = jnp.where(cntv == 1.0, W_DIRECT, W_MULTIPLE).astype(jnp.float32)
        cost0 = jnp.sum(jnp.where(validm, jnp.sqrt(d2_v[...]) * wgt, 0.0))
        pairs = jnp.where(validm, idx_v[...], -1)
        nempty = jnp.sum(jnp.where((cntv <= 0.0) & (miota < M), 1, 0))

        def bg_body(j, st):
            availf, prs, cst = st
            cj = cnt_s[j]
            vxj = vx_s[j]
            vyj = vy_s[j]
            dx = ux - vxj
            dy = uy - vyj
            ds2 = dx * dx + dy * dy
            cand = jnp.where(availf > 0.0, ds2, BIG)
            mv = jnp.min(cand)
            do = (cj == 0.0) & (mv < BIGH)
            mi = jnp.min(jnp.where(cand == mv, nidx, IBIG))
            prs = jnp.where((miota == j) & do, mi, prs)
            cst = cst + jnp.where(do, jnp.sqrt(mv) * W_BACKGROUND, 0.0)
            availf = jnp.where((nidx == mi) & do, 0.0, availf)
            return (availf, prs, cst)

        st = (avail, pairs, jnp.float32(0.0))
        _, pairs, bgcost = lax.cond(
            nempty > 0, lambda s: lax.fori_loop(0, M, bg_body, s),
            lambda s: s, st)

        pairs_o[...] = pairs
        cost_o[...] = (cost0 + bgcost).reshape(1, 1)

    smem = pl.BlockSpec(memory_space=pltpu.SMEM)
    vmem = pl.BlockSpec(memory_space=pltpu.VMEM)
    return pl.pallas_call(
        tc_body,
        in_specs=[smem, smem, smem, vmem, vmem, vmem, vmem, vmem, vmem],
        out_specs=[vmem, vmem],
        out_shape=[
            jax.ShapeDtypeStruct((1, MPAD), jnp.int32),
            jax.ShapeDtypeStruct((1, 1), jnp.float32),
        ],
    )


def kernel(pred_points, points, masks):
    N = pred_points.shape[0]
    M, H, W = masks.shape

    NPAD = ((N + NW * L - 1) // (NW * L)) * (NW * L)
    G = (M + L - 1) // L
    MG = G * L
    MPAD = 256
    NROW = 8
    KSHIFT = 6
    PPW = NPAD // NW

    ux = jnp.pad(pred_points[:, 0], (0, NPAD - N))
    uy = jnp.pad(pred_points[:, 1], (0, NPAD - N))
    v2 = jnp.pad(points, ((0, MPAD - M), (0, 0)))
    vgx = v2[:MG, 0].reshape(G, L).astype(jnp.int32)
    vgy = v2[:MG, 1].reshape(G, L).astype(jnp.int32)

    GW = L
    pack = _make_pack_kernel(M, H, W, GW)
    packed, anyg = pack(masks)
    packed = packed.reshape(-1)

    sc = _make_sc_kernel(N, NPAD, MG, H, W, G, GW)
    keys3, cnt3, anyp = sc(packed, anyg, ux, uy, vgx, vgy)

    keys2 = keys3.transpose(0, 2, 1).reshape(NW * L, MG)
    cnt2 = cnt3.transpose(0, 2, 1).reshape(NW * L, MG)

    red = _make_tc_reduce(N, NPAD, MG, MPAD, NROW, KSHIFT, PPW)
    cntv, d2v, idxv, avail = red(
        keys2, cnt2, anyp.reshape(NROW, NPAD // NROW))

    tc = _make_tc_matcher(N, NPAD, M, MPAD, NROW)
    pairs_v, cost = tc(
        cntv.reshape(MPAD), v2[:, 0], v2[:, 1],
        cntv, d2v, idxv, avail,
        ux.reshape(NROW, NPAD // NROW), uy.reshape(NROW, NPAD // NROW))

    pairs_arr = jnp.stack(
        [pairs_v[0, :M], jnp.arange(M, dtype=jnp.int32)], axis=1)
    return (pairs_arr, cost[0, 0])


# speedup vs baseline: 2.5902x; 1.1080x over previous
import functools

import jax
import jax.numpy as jnp
from jax import lax
from jax.experimental import pallas as pl
from jax.experimental.pallas import tpu as pltpu
from jax.experimental.pallas import tpu_sc as plsc

W_DIRECT = 1.0
W_MULTIPLE = 1.0
W_BACKGROUND = 1.0

NC = 2
NS = 16
L = 16
NW = NC * NS

BIG = 1e30
BIGH = 1e29
IBIG = 2**30


def _make_pack_kernel(M, H, W, GW):
    def pack_body(m_ref, p_ref, a_ref):
        rows = lax.broadcasted_iota(jnp.int32, (GW, M), 0)
        cols = lax.broadcasted_iota(jnp.int32, (GW, M), 1)
        sel = (cols >= rows * L) & (cols < rows * L + L)
        sh = jnp.clip(cols - rows * L, 0, L - 1)
        wmat = jnp.where(
            sel, jnp.left_shift(1, sh).astype(jnp.float32), 0.0)
        for r in range(8):
            acc = jax.lax.dot_general(
                wmat, m_ref[:, r, :], (((1,), (0,)), ((), ())),
                preferred_element_type=jnp.float32,
                precision=jax.lax.Precision.DEFAULT)
            anyrow = jnp.where(jnp.sum(acc, axis=0) > 0.0, 1.0, 0.0)
            p_ref[:, r * W:(r + 1) * W] = acc
            a_ref[r * W:(r + 1) * W] = anyrow

    return pl.pallas_call(
        pack_body,
        grid=(H // 8,),
        in_specs=[pl.BlockSpec((M, 8, W), lambda y: (0, y, 0))],
        out_specs=[
            pl.BlockSpec((GW, 8 * W), lambda y: (0, y)),
            pl.BlockSpec((8 * W,), lambda y: (y,)),
        ],
        out_shape=[
            jax.ShapeDtypeStruct((GW, H * W), jnp.float32),
            jax.ShapeDtypeStruct((H * W,), jnp.float32),
        ],
    )


def _make_sc_kernel(N, NPAD, MG, H, W, G, GW):
    HW = H * W
    WSHIFT = W.bit_length() - 1
    assert W == (1 << WSHIFT)
    PPW = NPAD // NW
    PCH = PPW // L
    assert PCH <= 64
    KSHIFT = 6
    mesh = plsc.VectorSubcoreMesh(core_axis_name="c", subcore_axis_name="s")

    @functools.partial(
        pl.kernel,
        mesh=mesh,
        compiler_params=pltpu.CompilerParams(needs_layout_passes=False),
        out_type=[
            jax.ShapeDtypeStruct((NW, MG, L), jnp.int32),
            jax.ShapeDtypeStruct((NW, MG, L), jnp.int32),
            jax.ShapeDtypeStruct((NPAD,), jnp.float32),
        ],
        scratch_types=[
            pltpu.VMEM((PPW,), jnp.float32),
            pltpu.VMEM((PPW,), jnp.float32),
            pltpu.VMEM((PPW,), jnp.int32),
            pltpu.VMEM((PPW,), jnp.int32),
            pltpu.VMEM((PPW,), jnp.int32),
            pltpu.VMEM((PPW,), jnp.float32),
            pltpu.VMEM((PPW,), jnp.float32),
            pltpu.VMEM((PPW,), jnp.float32),
            pltpu.VMEM((PPW,), jnp.int32),
            pltpu.VMEM((PPW,), jnp.int32),
            pltpu.VMEM((G, L), jnp.int32),
            pltpu.VMEM((G, L), jnp.int32),
            pltpu.VMEM((MG, L), jnp.int32),
            pltpu.VMEM((MG, L), jnp.int32),
            pltpu.SemaphoreType.DMA,
            pltpu.SemaphoreType.DMA,
        ],
    )
    def sc_kernel(packed_hbm, anyg_hbm, ux_hbm, uy_hbm, vx_hbm, vy_hbm,
                  keys_o, cnt_o, any_o,
                  tmpx, tmpy, pidx_v, offs0_v, offs1_v, col0_v, col1_v,
                  anyv_v, xv, yv, vgx, vgy, keysb, cntb, sem0, sem1):
        wid = lax.axis_index("s") * NC + lax.axis_index("c")
        base = wid * PPW
        pltpu.sync_copy(ux_hbm.at[pl.ds(base, PPW)], tmpx)
        pltpu.sync_copy(uy_hbm.at[pl.ds(base, PPW)], tmpy)
        pltpu.sync_copy(vx_hbm, vgx)
        pltpu.sync_copy(vy_hbm, vgy)

        def init_body(i):
            sl = pl.ds(i * L, L)
            xi = jnp.clip(tmpx[sl].astype(jnp.int32), 0, W - 1)
            yi = jnp.clip(tmpy[sl].astype(jnp.int32), 0, H - 1)
            xv[sl] = xi
            yv[sl] = yi
            pidx_v[sl] = yi * W + xi
        plsc.parallel_loop(0, PCH, unroll=8)(init_body)

        pltpu.async_copy(anyg_hbm.at[pidx_v], anyv_v, sem1).wait()

        nreal = jnp.maximum(jnp.minimum(N - base, PPW), 0)
        biota = lax.iota(jnp.int32, L)

        def pad_any(i, c):
            sl = pl.ds(i * L, L)
            pt = biota + i * L
            anyv_v[sl] = jnp.where(pt < nreal, anyv_v[sl], 1.0)
            return c
        lax.fori_loop(nreal // L, PCH, pad_any, 0)

        pltpu.sync_copy(anyv_v, any_o.at[pl.ds(base, PPW)])

        base_iota = lax.iota(jnp.int32, L)

        offs_bufs = (offs0_v, offs1_v)
        col_bufs = (col0_v, col1_v)
        sems = (sem0, sem1)
        GP = (G + 1) // 2

        def issue(g, b):
            goff = g * HW

            def offs_body(i):
                sl = pl.ds(i * L, L)
                offs_bufs[b][sl] = pidx_v[sl] + goff
            plsc.parallel_loop(0, PCH, unroll=8)(offs_body)
            pltpu.async_copy(packed_hbm.at[offs_bufs[b]], col_bufs[b],
                             sems[b])

        def process(g, b):
            colv = col_bufs[b]
            pltpu.make_async_copy(
                packed_hbm.at[offs_bufs[b]], colv, sems[b]).wait()

            def padc(i, c):
                sl = pl.ds(i * L, L)
                pt = biota + i * L
                colv[sl] = jnp.where(pt < nreal, colv[sl], 0.0)
                return c
            lax.fori_loop(nreal // L, PCH, padc, 0)

            vgxr = vgx[g, :]
            vgyr = vgy[g, :]
            for j in range(L):
                vxj = vgxr[j]
                vyj = vgyr[j]

                def chunk_body(c, cr):
                    bkey, cnt = cr
                    sl = pl.ds(c * L, L)
                    wi = colv[sl].astype(jnp.int32)
                    bit = jnp.bitwise_and(
                        lax.shift_right_logical(wi, j), 1)
                    dx = xv[sl] - vxj
                    dy = yv[sl] - vyj
                    d2 = dx * dx + dy * dy
                    key = jnp.bitwise_or(lax.shift_left(d2, KSHIFT), c)
                    key = jnp.where(bit > 0, key, IBIG)
                    bkey = jnp.minimum(bkey, key)
                    cnt = cnt + bit
                    return (bkey, cnt)

                bkey, cnt = plsc.parallel_loop(
                    0, PCH, unroll=8,
                    carry=(jnp.full((L,), IBIG, jnp.int32),
                           jnp.zeros((L,), jnp.int32)),
                )(chunk_body)
                keysb[g * L + j, :] = bkey
                cntb[g * L + j, :] = cnt

        issue(0, 0)
        issue(1, 1)

        def pair_body(t, carry):
            g0 = 2 * t
            g1 = 2 * t + 1
            process(g0, 0)

            @pl.when(g0 + 2 < G)
            def _():
                issue(g0 + 2, 0)

            @pl.when(g1 < G)
            def _():
                process(g1, 1)

                @pl.when(g1 + 2 < G)
                def _():
                    issue(g1 + 2, 1)
            return carry
        lax.fori_loop(0, GP, pair_body, 0)

        pltpu.sync_copy(keysb, keys_o.at[wid])
        pltpu.sync_copy(cntb, cnt_o.at[wid])

    return sc_kernel


def _make_tc_reduce(N, NPAD, MG, MPAD, NROW, KSHIFT, PPW):
    NCOL = NPAD // NROW

    def red_body(keys_r, cnt_r, any_r, cntv_o, d2v_o, idxv_o, avail_o):
        keys = keys_r[...]
        cnts = cnt_r[...]
        i0 = lax.broadcasted_iota(jnp.int32, (NW * L, MG), 0)
        w = i0 // L
        lane = i0 - w * L
        d2 = lax.shift_right_logical(keys, KSHIFT)
        c = jnp.bitwise_and(keys, (1 << KSHIFT) - 1)
        gidx = w * PPW + c * L + lane
        mind2 = jnp.min(d2, axis=0, keepdims=True)
        idxc = jnp.where(d2 == mind2, gidx, IBIG)
        minidx = jnp.min(idxc, axis=0, keepdims=True)
        cntm = jnp.sum(cnts, axis=0, keepdims=True)

        pad = jnp.zeros((1, MPAD - MG), jnp.float32)
        ipad = jnp.zeros((1, MPAD - MG), jnp.int32)
        cntv_o[...] = jnp.concatenate(
            [cntm.astype(jnp.float32), pad], axis=1)
        d2v_o[...] = jnp.concatenate(
            [mind2.astype(jnp.float32), pad], axis=1)
        idxv_o[...] = jnp.concatenate([minidx, ipad], axis=1)

        anyv = any_r[...]
        rr = lax.broadcasted_iota(jnp.int32, (NROW, NCOL), 0)
        cc = lax.broadcasted_iota(jnp.int32, (NROW, NCOL), 1)
        nidx = rr * NCOL + cc
        avail_o[...] = jnp.where((anyv == 0.0) & (nidx < N), 1.0, 0.0)

    vmem = pl.BlockSpec(memory_space=pltpu.VMEM)
    return pl.pallas_call(
        red_body,
        in_specs=[vmem, vmem, vmem],
        out_specs=[vmem, vmem, vmem, vmem],
        out_shape=[
            jax.ShapeDtypeStruct((1, MPAD), jnp.float32),
            jax.ShapeDtypeStruct((1, MPAD), jnp.float32),
            jax.ShapeDtypeStruct((1, MPAD), jnp.int32),
            jax.ShapeDtypeStruct((NROW, NCOL), jnp.float32),
        ],
    )


def _make_tc_matcher(N, NPAD, M, MPAD, NROW):
    NCOL = NPAD // NROW

    def tc_body(cnt_s, vx_s, vy_s, cnt_v, d2_v, idx_v, avail_r, ux_r, uy_r,
                pairs_o, cost_o):
        ux = ux_r[...]
        uy = uy_r[...]
        rr = lax.broadcasted_iota(jnp.int32, (NROW, NCOL), 0)
        cc = lax.broadcasted_iota(jnp.int32, (NROW, NCOL), 1)
        nidx = rr * NCOL + cc
        avail = avail_r[...]

        miota = lax.broadcasted_iota(jnp.int32, (1, MPAD), 1)
        cntv = cnt_v[...]
        validm = (cntv > 0.0) & (miota < M)
        wgt = jnp.where(cntv == 1.0, W_DIRECT, W_MULTIPLE).astype(jnp.float32)
        cost0 = jnp.sum(jnp.where(validm, jnp.sqrt(d2_v[...]) * wgt, 0.0))
        pairs = jnp.where(validm, idx_v[...], -1)
        nempty = jnp.sum(jnp.where((cntv <= 0.0) & (miota < M), 1, 0))

        def bg_body(j, st):
            availf, prs, cst = st
            cj = cnt_s[j]
            vxj = vx_s[j]
            vyj = vy_s[j]
            dx = ux - vxj
            dy = uy - vyj
            ds2 = dx * dx + dy * dy
            cand = jnp.where(availf > 0.0, ds2, BIG)
            mv = jnp.min(cand)
            do = (cj == 0.0) & (mv < BIGH)
            mi = jnp.min(jnp.where(cand == mv, nidx, IBIG))
            prs = jnp.where((miota == j) & do, mi, prs)
            cst = cst + jnp.where(do, jnp.sqrt(mv) * W_BACKGROUND, 0.0)
            availf = jnp.where((nidx == mi) & do, 0.0, availf)
            return (availf, prs, cst)

        st = (avail, pairs, jnp.float32(0.0))
        _, pairs, bgcost = lax.cond(
            nempty > 0, lambda s: lax.fori_loop(0, M, bg_body, s),
            lambda s: s, st)

        pairs_o[...] = pairs
        cost_o[...] = (cost0 + bgcost).reshape(1, 1)

    smem = pl.BlockSpec(memory_space=pltpu.SMEM)
    vmem = pl.BlockSpec(memory_space=pltpu.VMEM)
    return pl.pallas_call(
        tc_body,
        in_specs=[smem, smem, smem, vmem, vmem, vmem, vmem, vmem, vmem],
        out_specs=[vmem, vmem],
        out_shape=[
            jax.ShapeDtypeStruct((1, MPAD), jnp.int32),
            jax.ShapeDtypeStruct((1, 1), jnp.float32),
        ],
    )


def kernel(pred_points, points, masks):
    N = pred_points.shape[0]
    M, H, W = masks.shape

    NPAD = ((N + NW * L - 1) // (NW * L)) * (NW * L)
    G = (M + L - 1) // L
    MG = G * L
    MPAD = 256
    NROW = 8
    KSHIFT = 6
    PPW = NPAD // NW

    ux = jnp.pad(pred_points[:, 0], (0, NPAD - N))
    uy = jnp.pad(pred_points[:, 1], (0, NPAD - N))
    v2 = jnp.pad(points, ((0, MPAD - M), (0, 0)))
    vgx = v2[:MG, 0].reshape(G, L).astype(jnp.int32)
    vgy = v2[:MG, 1].reshape(G, L).astype(jnp.int32)

    GW = L
    pack = _make_pack_kernel(M, H, W, GW)
    packed, anyg = pack(masks)
    packed = packed.reshape(-1)

    sc = _make_sc_kernel(N, NPAD, MG, H, W, G, GW)
    keys3, cnt3, anyp = sc(packed, anyg, ux, uy, vgx, vgy)

    keys2 = keys3.transpose(0, 2, 1).reshape(NW * L, MG)
    cnt2 = cnt3.transpose(0, 2, 1).reshape(NW * L, MG)

    red = _make_tc_reduce(N, NPAD, MG, MPAD, NROW, KSHIFT, PPW)
    cntv, d2v, idxv, avail = red(
        keys2, cnt2, anyp.reshape(NROW, NPAD // NROW))

    tc = _make_tc_matcher(N, NPAD, M, MPAD, NROW)
    pairs_v, cost = tc(
        cntv.reshape(MPAD), v2[:, 0], v2[:, 1],
        cntv, d2v, idxv, avail,
        ux.reshape(NROW, NPAD // NROW), uy.reshape(NROW, NPAD // NROW))

    pairs_arr = jnp.stack(
        [pairs_v[0, :M], jnp.arange(M, dtype=jnp.int32)], axis=1)
    return (pairs_arr, cost[0, 0])
